# baseline (speedup 1.0000x reference)
import jax
import jax.numpy as jnp
from jax import lax
from jax.experimental import pallas as pl
from jax.experimental.pallas import tpu as pltpu
from jax.experimental.pallas import tpu_sc as plsc

B = 16384
D = 64
L = 16
NC, NS = 2, 16
NW = NC * NS
BPW = B // NW
K = 128
NCHUNK = BPW // K


def _body(iridx_hbm, ihalf_hbm, uridx_hbm, uhalf_hbm, it_hbm, ut_hbm,
          out_hbm, iidx_v, ihv_v, uidx_v, uhv_v, irows_v, urows_v, out_v):
    wid = lax.axis_index("s") * NC + lax.axis_index("c")
    base = wid * BPW
    pltpu.sync_copy(iridx_hbm.at[pl.ds(base, BPW)], iidx_v)
    pltpu.sync_copy(ihalf_hbm.at[pl.ds(base, BPW)], ihv_v)
    pltpu.sync_copy(uridx_hbm.at[pl.ds(base, BPW)], uidx_v)
    pltpu.sync_copy(uhalf_hbm.at[pl.ds(base, BPW)], uhv_v)

    lane = lax.iota(jnp.int32, L)

    for c in range(NCHUNK):
        pltpu.sync_copy(it_hbm.at[iidx_v.at[pl.ds(c * K, K)]], irows_v)
        pltpu.sync_copy(ut_hbm.at[uidx_v.at[pl.ds(c * K, K)]], urows_v)

        @pl.loop(0, K // L)
        def _(g):
            o = jnp.zeros((L,), jnp.float32)
            ih16 = ihv_v[pl.ds(c * K + g * L, L)]
            uh16 = uhv_v[pl.ds(c * K + g * L, L)]
            for r in range(L):
                row = g * L + r
                ioff = ih16[r] * D
                uoff = uh16[r] * D
                s16 = (irows_v[row, pl.ds(ioff, L)]
                       * urows_v[row, pl.ds(uoff, L)])
                for j in range(1, D // L):
                    s16 = s16 + (irows_v[row, pl.ds(ioff + j * L, L)]
                                 * urows_v[row, pl.ds(uoff + j * L, L)])
                o = jnp.where(lane == r, jnp.sum(s16), o)
            out_v[pl.ds(c * K + g * L, L)] = o

    pltpu.sync_copy(out_v, out_hbm.at[pl.ds(base, BPW)])


def kernel(item_vec, user_vec, item_table, user_table):
    iridx = jax.lax.shift_right_logical(item_vec, 1)
    uridx = jax.lax.shift_right_logical(user_vec, 1)
    ihalf = jax.lax.bitwise_and(item_vec, 1)
    uhalf = jax.lax.bitwise_and(user_vec, 1)
    it2 = item_table.reshape(item_table.shape[0] // 2, 2 * D)
    ut2 = user_table.reshape(user_table.shape[0] // 2, 2 * D)

    mesh = plsc.VectorSubcoreMesh(core_axis_name="c", subcore_axis_name="s")
    cp = pltpu.CompilerParams(
        needs_layout_passes=False, use_tc_tiling_on_sc=False)
    f = pl.kernel(
        _body,
        out_type=jax.ShapeDtypeStruct((B,), jnp.float32),
        mesh=mesh,
        compiler_params=cp,
        scratch_types=[
            pltpu.VMEM((BPW,), jnp.int32),
            pltpu.VMEM((BPW,), jnp.int32),
            pltpu.VMEM((BPW,), jnp.int32),
            pltpu.VMEM((BPW,), jnp.int32),
            pltpu.VMEM((K, 2 * D), jnp.float32),
            pltpu.VMEM((K, 2 * D), jnp.float32),
            pltpu.VMEM((BPW,), jnp.float32),
        ],
    )
    return f(iridx, ihalf, uridx, uhalf, it2, ut2)


# speedup vs baseline: 1.0029x; 1.0029x over previous
import jax
import jax.numpy as jnp
from jax import lax
from jax.experimental import pallas as pl
from jax.experimental.pallas import tpu as pltpu
from jax.experimental.pallas import tpu_sc as plsc

B = 16384
D = 64
L = 16
NC, NS = 2, 16
NW = NC * NS
BPW = B // NW
K = 128
NCHUNK = BPW // K


def _body(iridx_hbm, ihalf_hbm, uridx_hbm, uhalf_hbm, it_hbm, ut_hbm,
          out_hbm, iidx_v, ihv_v, uidx_v, uhv_v, irows_v, urows_v, out_v):
    wid = lax.axis_index("s") * NC + lax.axis_index("c")
    base = wid * BPW
    pltpu.sync_copy(iridx_hbm.at[pl.ds(base, BPW)], iidx_v)
    pltpu.sync_copy(ihalf_hbm.at[pl.ds(base, BPW)], ihv_v)
    pltpu.sync_copy(uridx_hbm.at[pl.ds(base, BPW)], uidx_v)
    pltpu.sync_copy(uhalf_hbm.at[pl.ds(base, BPW)], uhv_v)

    lane = lax.iota(jnp.int32, L)

    for c in range(NCHUNK):
        pltpu.sync_copy(it_hbm.at[iidx_v.at[pl.ds(c * K, K)]], irows_v)
        pltpu.sync_copy(ut_hbm.at[uidx_v.at[pl.ds(c * K, K)]], urows_v)

        @pl.loop(0, K // L)
        def _(g):
            o = jnp.zeros((L,), jnp.float32)
            ih16 = ihv_v[pl.ds(c * K + g * L, L)]
            uh16 = uhv_v[pl.ds(c * K + g * L, L)]
            for r in range(L):
                row = g * L + r
                ioff = ih16[r] * D
                uoff = uh16[r] * D
                s16 = (irows_v[row, pl.ds(ioff, L)]
                       * urows_v[row, pl.ds(uoff, L)])
                for j in range(1, D // L):
                    s16 = s16 + (irows_v[row, pl.ds(ioff + j * L, L)]
                                 * urows_v[row, pl.ds(uoff + j * L, L)])
                o = jnp.where(lane == r, jnp.sum(s16), o)
            out_v[pl.ds(c * K + g * L, L)] = o

    pltpu.sync_copy(out_v, out_hbm.at[pl.ds(base, BPW)])


def kernel(item_vec, user_vec, item_table, user_table):
    iridx = jax.lax.shift_right_logical(item_vec, 1)
    uridx = jax.lax.shift_right_logical(user_vec, 1)
    ihalf = jax.lax.bitwise_and(item_vec, 1)
    uhalf = jax.lax.bitwise_and(user_vec, 1)
    it2 = item_table.reshape(item_table.shape[0] // 2, 2 * D)
    ut2 = user_table.reshape(user_table.shape[0] // 2, 2 * D)

    mesh = plsc.VectorSubcoreMesh(core_axis_name="c", subcore_axis_name="s")
    cp = pltpu.CompilerParams(
        needs_layout_passes=False, use_tc_tiling_on_sc=True)
    f = pl.kernel(
        _body,
        out_type=jax.ShapeDtypeStruct((B,), jnp.float32),
        mesh=mesh,
        compiler_params=cp,
        scratch_types=[
            pltpu.VMEM((BPW,), jnp.int32),
            pltpu.VMEM((BPW,), jnp.int32),
            pltpu.VMEM((BPW,), jnp.int32),
            pltpu.VMEM((BPW,), jnp.int32),
            pltpu.VMEM((K, 2 * D), jnp.float32),
            pltpu.VMEM((K, 2 * D), jnp.float32),
            pltpu.VMEM((BPW,), jnp.float32),
        ],
    )
    return f(iridx, ihalf, uridx, uhalf, it2, ut2)


# speedup vs baseline: 1.4235x; 1.4193x over previous
import jax
import jax.numpy as jnp
from jax import lax
from jax.experimental import pallas as pl
from jax.experimental.pallas import tpu as pltpu
from jax.experimental.pallas import tpu_sc as plsc

B = 16384
D = 64
L = 16
NC, NS = 2, 16
NW = NC * NS
BPW = B // NW
SL = 8


def _body(iv_hbm, uv_hbm, it_hbm, ut_hbm, out_hbm,
          iidx_v, uidx_v, it_tiles, ut_tiles, out_v, sem):
    wid = lax.axis_index("s") * NC + lax.axis_index("c")
    base = wid * BPW
    pltpu.sync_copy(iv_hbm.at[pl.ds(base, BPW)], iidx_v)
    pltpu.sync_copy(uv_hbm.at[pl.ds(base, BPW)], uidx_v)

    lane = lax.iota(jnp.int32, L)

    @pl.loop(0, BPW // L)
    def _(g):
        off = pl.multiple_of(g * L, L)
        iv16 = iidx_v[pl.ds(off, L)]
        uv16 = uidx_v[pl.ds(off, L)]
        ibase = (jax.lax.shift_right_logical(iv16, 3)) * SL
        ubase = (jax.lax.shift_right_logical(uv16, 3)) * SL

        copies = []
        for r in range(L):
            ib = pl.multiple_of(ibase[r], SL)
            ub = pl.multiple_of(ubase[r], SL)
            copies.append(pltpu.async_copy(
                it_hbm.at[pl.ds(ib, SL)], it_tiles.at[r], sem))
            copies.append(pltpu.async_copy(
                ut_hbm.at[pl.ds(ub, SL)], ut_tiles.at[r], sem))
        for cp in copies:
            cp.wait()

        isub = jax.lax.bitwise_and(iv16, SL - 1)
        usub = jax.lax.bitwise_and(uv16, SL - 1)
        acc = jnp.zeros((L,), jnp.float32)
        for c in range(D):
            cv = jnp.full((L,), c, jnp.int32)
            a = plsc.load_gather(it_tiles, [lane, isub, cv])
            b = plsc.load_gather(ut_tiles, [lane, usub, cv])
            acc = acc + a * b
        out_v[pl.ds(off, L)] = acc

    pltpu.sync_copy(out_v, out_hbm.at[pl.ds(base, BPW)])


def kernel(item_vec, user_vec, item_table, user_table):
    mesh = plsc.VectorSubcoreMesh(core_axis_name="c", subcore_axis_name="s")
    f = pl.kernel(
        _body,
        out_type=jax.ShapeDtypeStruct((B,), jnp.float32),
        mesh=mesh,
        compiler_params=pltpu.CompilerParams(needs_layout_passes=False),
        scratch_types=[
            pltpu.VMEM((BPW,), jnp.int32),
            pltpu.VMEM((BPW,), jnp.int32),
            pltpu.VMEM((L, SL, D), jnp.float32),
            pltpu.VMEM((L, SL, D), jnp.float32),
            pltpu.VMEM((BPW,), jnp.float32),
            pltpu.SemaphoreType.DMA,
        ],
    )
    return f(item_vec, user_vec, item_table, user_table)
